# baseline (speedup 1.0000x reference)
import jax
import jax.numpy as jnp
from jax.experimental import pallas as pl
from jax.experimental.pallas import tpu as pltpu

_K = 8
_G = 32
_EPS = 1e-8


def _topk_body(s_ref, out_ref):
    s = s_ref[...]
    neg = jnp.finfo(jnp.float32).min
    iota = jax.lax.broadcasted_iota(jnp.int32, s.shape, 1)
    idxs = []
    for _ in range(_K):
        idx = jnp.argmax(s, axis=1).astype(jnp.int32)
        idxs.append(idx)
        s = jnp.where(iota == idx[:, None], neg, s)
    out_ref[...] = jnp.stack(idxs, axis=1)


def _gather_body(topk_ref, neigh_ref, out_ref):
    del topk_ref
    out_ref[...] = neigh_ref[...].reshape(out_ref.shape)


def _select_body(cand_ref, logits_ref, out_ref, mask_ref):
    logits = logits_ref[...]
    cand = cand_ref[...]
    br, n_senses = logits.shape
    iota = jax.lax.broadcasted_iota(jnp.int32, (br, n_senses), 1)
    cand_iota = jax.lax.broadcasted_iota(jnp.int32, cand.shape, 1)

    tile = 2048
    for start in range(0, n_senses, tile):
        w = min(tile, n_senses - start)
        tiota = jax.lax.broadcasted_iota(jnp.int32, (br, w), 1) + start

        def accum(c, m):
            col = jnp.sum(jnp.where(cand_iota == c, cand, 0), axis=1,
                          keepdims=True)
            return m | (tiota == col).astype(jnp.int32)

        mtile = jax.lax.fori_loop(0, _K * _G, accum,
                                  jnp.zeros((br, w), jnp.int32))
        mask_ref[:, start:start + w] = mtile

    mask = mask_ref[...] > 0
    neg = jnp.finfo(jnp.float32).min
    ml = jnp.where(mask, logits, neg)
    rowmax = jnp.max(ml, axis=1, keepdims=True)
    e = jnp.where(mask, jnp.exp(ml - rowmax), 0.0)
    sm = e / jnp.sum(e, axis=1, keepdims=True)
    n_sel = jnp.sum(mask.astype(jnp.float32), axis=1, keepdims=True)
    delta = _EPS * (n_senses - n_sel)
    amax = jnp.argmax(sm, axis=1).astype(jnp.int32)
    sm = sm - delta * (iota == amax[:, None]).astype(jnp.float32)
    out_ref[...] = jnp.log(jnp.where(mask, sm, _EPS))


def kernel(scores, logits_senses, neighbours):
    n, vocab = scores.shape
    _, n_senses = logits_senses.shape
    br = 16

    topk_idx = pl.pallas_call(
        _topk_body,
        grid=(n // br,),
        in_specs=[pl.BlockSpec((br, vocab), lambda i: (i, 0))],
        out_specs=pl.BlockSpec((br, _K), lambda i: (i, 0)),
        out_shape=jax.ShapeDtypeStruct((n, _K), jnp.int32),
    )(scores)

    cand = pl.pallas_call(
        _gather_body,
        grid_spec=pltpu.PrefetchScalarGridSpec(
            num_scalar_prefetch=1,
            grid=(_K, n),
            in_specs=[pl.BlockSpec(
                (1, 1, _G), lambda k, r, topk: (topk[r, k], 0, 0))],
            out_specs=pl.BlockSpec(
                (1, 1, 1, _G), lambda k, r, topk: (r, k, 0, 0)),
        ),
        out_shape=jax.ShapeDtypeStruct((n, _K, 1, _G), jnp.int32),
    )(topk_idx, neighbours.reshape(vocab, 1, _G))
    cand = cand.reshape(n, _K * _G)

    return pl.pallas_call(
        _select_body,
        grid=(n // br,),
        in_specs=[
            pl.BlockSpec((br, _K * _G), lambda i: (i, 0)),
            pl.BlockSpec((br, n_senses), lambda i: (i, 0)),
        ],
        out_specs=pl.BlockSpec((br, n_senses), lambda i: (i, 0)),
        out_shape=jax.ShapeDtypeStruct((n, n_senses), jnp.float32),
        scratch_shapes=[pltpu.VMEM((br, n_senses), jnp.int32)],
    )(cand, logits_senses)


# speedup vs baseline: 3.5087x; 3.5087x over previous
import jax
import jax.numpy as jnp
from jax.experimental import pallas as pl
from jax.experimental.pallas import tpu as pltpu

_K = 8
_G = 32
_EPS = 1e-8


def _topk_body(s_ref, out_ref):
    s = s_ref[...]
    neg = jnp.finfo(jnp.float32).min
    big = jnp.iinfo(jnp.int32).max
    iota = jax.lax.broadcasted_iota(jnp.int32, s.shape, 1)
    idxs = []
    for _ in range(_K):
        m = jnp.max(s, axis=1, keepdims=True)
        idx = jnp.min(jnp.where(s == m, iota, big), axis=1)
        idxs.append(idx)
        s = jnp.where(iota == idx[:, None], neg, s)
    out_ref[...] = jnp.stack(idxs, axis=1)


def _gather_body(topk_ref, neigh_ref, out_ref):
    del topk_ref
    out_ref[...] = neigh_ref[...].reshape(out_ref.shape)


def _select_body(cand_ref, logits_ref, out_ref, mask_ref):
    logits = logits_ref[...]
    cand = cand_ref[...]
    br, n_senses = logits.shape
    iota = jax.lax.broadcasted_iota(jnp.int32, (br, n_senses), 1)
    cand_iota = jax.lax.broadcasted_iota(jnp.int32, cand.shape, 1)

    chunk = 16
    mask_ref[...] = jnp.zeros((br, n_senses), jnp.int32)

    def accum(i, _):
        acc = None
        for u in range(chunk):
            c = i * chunk + u
            col = jnp.sum(jnp.where(cand_iota == c, cand, 0), axis=1,
                          keepdims=True)
            eq = iota == col
            acc = eq if acc is None else (acc | eq)
        mask_ref[...] = mask_ref[...] | acc.astype(jnp.int32)
        return 0

    jax.lax.fori_loop(0, (_K * _G) // chunk, accum, 0)
    mask = mask_ref[...] > 0
    neg = jnp.finfo(jnp.float32).min
    ml = jnp.where(mask, logits, neg)
    rowmax = jnp.max(ml, axis=1, keepdims=True)
    e = jnp.where(mask, jnp.exp(ml - rowmax), 0.0)
    sm = e / jnp.sum(e, axis=1, keepdims=True)
    n_sel = jnp.sum(mask.astype(jnp.float32), axis=1, keepdims=True)
    delta = _EPS * (n_senses - n_sel)
    smax = jnp.max(sm, axis=1, keepdims=True)
    amax = jnp.min(jnp.where(sm == smax, iota, jnp.iinfo(jnp.int32).max),
                   axis=1)
    sm = sm - delta * (iota == amax[:, None]).astype(jnp.float32)
    out_ref[...] = jnp.log(jnp.where(mask, sm, _EPS))


def kernel(scores, logits_senses, neighbours):
    n, vocab = scores.shape
    _, n_senses = logits_senses.shape
    br = 16

    topk_idx = pl.pallas_call(
        _topk_body,
        grid=(n // br,),
        in_specs=[pl.BlockSpec((br, vocab), lambda i: (i, 0))],
        out_specs=pl.BlockSpec((br, _K), lambda i: (i, 0)),
        out_shape=jax.ShapeDtypeStruct((n, _K), jnp.int32),
    )(scores)

    cand = pl.pallas_call(
        _gather_body,
        grid_spec=pltpu.PrefetchScalarGridSpec(
            num_scalar_prefetch=1,
            grid=(_K, n),
            in_specs=[pl.BlockSpec(
                (1, 1, _G), lambda k, r, topk: (topk[r, k], 0, 0))],
            out_specs=pl.BlockSpec(
                (1, 1, 1, _G), lambda k, r, topk: (r, k, 0, 0)),
        ),
        out_shape=jax.ShapeDtypeStruct((n, _K, 1, _G), jnp.int32),
    )(topk_idx, neighbours.reshape(vocab, 1, _G))
    cand = cand.reshape(n, _K * _G)

    return pl.pallas_call(
        _select_body,
        grid=(n // br,),
        in_specs=[
            pl.BlockSpec((br, _K * _G), lambda i: (i, 0)),
            pl.BlockSpec((br, n_senses), lambda i: (i, 0)),
        ],
        out_specs=pl.BlockSpec((br, n_senses), lambda i: (i, 0)),
        out_shape=jax.ShapeDtypeStruct((n, n_senses), jnp.float32),
        scratch_shapes=[pltpu.VMEM((br, n_senses), jnp.int32)],
    )(cand, logits_senses)
